# Initial kernel scaffold; baseline (speedup 1.0000x reference)
import jax
import jax.numpy as jnp
from jax.experimental import pallas as pl


def kernel(hidden_states, W_router, wi, wo):
    raise NotImplementedError("write your pallas kernel here")


# speedup vs baseline: 1.8919x; 1.8919x over previous
import functools

import jax
import jax.numpy as jnp
from jax import lax
from jax.experimental import pallas as pl
from jax.experimental.pallas import tpu as pltpu
from jax.experimental.pallas import tpu_sc as plsc

_T = 256
_NW = 32


def _router_body(x_ref, w_ref, logits_ref, idx_ref, xs_ref):
    logits = jnp.dot(x_ref[...], w_ref[...], preferred_element_type=jnp.float32)
    logits_ref[...] = logits
    m = jnp.max(logits, axis=-1, keepdims=True)
    s = jnp.sum(jnp.exp(logits - m), axis=-1, keepdims=True)
    pmax = 1.0 / s
    e = logits.shape[-1]
    lane = lax.broadcasted_iota(jnp.int32, logits.shape, 1)
    idx_ref[...] = jnp.min(jnp.where(logits == m, lane, e), axis=-1,
                           keepdims=True)
    xs_ref[...] = x_ref[...] * pmax


def _route(x, w_router, interpret=False):
    s, d = x.shape
    e = w_router.shape[-1]
    return pl.pallas_call(
        _router_body,
        out_shape=[
            jax.ShapeDtypeStruct((s, e), jnp.float32),
            jax.ShapeDtypeStruct((s, 1), jnp.int32),
            jax.ShapeDtypeStruct((s, d), jnp.float32),
        ],
        interpret=interpret,
    )(x, w_router)


def _dispatch_meta(eidx, num_experts, num_tiles):
    s = eidx.shape[0]
    oh = jax.nn.one_hot(eidx, num_experts, dtype=jnp.int32)
    csum = jnp.cumsum(oh, axis=0)
    counts = csum[-1]
    ends = jnp.cumsum(counts)
    starts = ends - counts
    within = jnp.sum((csum - oh) * oh, axis=1)
    pos = (starts[eidx] + within).astype(jnp.int32)

    n_items = num_tiles + num_experts - 1
    tfirst = jnp.arange(num_tiles, dtype=jnp.int32) * _T
    ef = jnp.searchsorted(ends, tfirst, side="right").astype(jnp.int32)
    el = jnp.searchsorted(ends, tfirst + (_T - 1), side="right").astype(jnp.int32)
    cnt = el - ef + 1
    base = jnp.cumsum(cnt) - cnt
    total = base[-1] + cnt[-1]
    wi_ = jnp.arange(n_items, dtype=jnp.int32)
    tw = jnp.clip(jnp.searchsorted(base, wi_, side="right") - 1,
                  0, num_tiles - 1).astype(jnp.int32)
    k = wi_ - base[tw]
    ew = jnp.clip(ef[tw] + k, 0, num_experts - 1).astype(jnp.int32)
    valid = wi_ < total
    lo = jnp.clip(starts[ew] - tw * _T, 0, _T)
    hi = jnp.clip(ends[ew] - tw * _T, 0, _T)
    lo = jnp.where(valid, lo, 0).astype(jnp.int32)
    hi = jnp.where(valid, hi, 0).astype(jnp.int32)
    first = (valid & (k == 0)).astype(jnp.int32)
    return pos, tw, ew, lo, hi, first


def _ffn_body(tw_ref, ew_ref, lo_ref, hi_ref, first_ref,
              xs_ref, wi_ref, wo_ref, y_ref):
    w = pl.program_id(0)

    @pl.when(first_ref[w] == 1)
    def _init():
        y_ref[...] = jnp.zeros_like(y_ref)

    h = jnp.maximum(
        jnp.dot(xs_ref[...], wi_ref[0], preferred_element_type=jnp.float32),
        0.0)
    y = jnp.dot(h, wo_ref[0], preferred_element_type=jnp.float32)
    rows = lax.broadcasted_iota(jnp.int32, (y.shape[0], 1), 0)
    mask = (rows >= lo_ref[w]) & (rows < hi_ref[w])
    y_ref[...] += jnp.where(mask, y, 0.0)


def _grouped_ffn(xs, wi, wo, meta, interpret=False):
    s, d = xs.shape
    e, _, f = wi.shape
    tw, ew, lo, hi, first = meta
    n_items = tw.shape[0]
    grid_spec = pltpu.PrefetchScalarGridSpec(
        num_scalar_prefetch=5,
        grid=(n_items,),
        in_specs=[
            pl.BlockSpec((_T, d), lambda w, tw, ew, lo, hi, fs: (tw[w], 0)),
            pl.BlockSpec((1, d, f), lambda w, tw, ew, lo, hi, fs: (ew[w], 0, 0)),
            pl.BlockSpec((1, f, d), lambda w, tw, ew, lo, hi, fs: (ew[w], 0, 0)),
        ],
        out_specs=pl.BlockSpec((_T, d), lambda w, tw, ew, lo, hi, fs: (tw[w], 0)),
    )
    return pl.pallas_call(
        _ffn_body,
        grid_spec=grid_spec,
        out_shape=jax.ShapeDtypeStruct((s, d), jnp.float32),
        compiler_params=pltpu.CompilerParams(
            dimension_semantics=("arbitrary",)),
        interpret=interpret,
    )(tw, ew, lo, hi, first, xs, wi, wo)


def _sc_scatter_rows(src, pos):
    s, d = src.shape
    rpw = s // _NW
    mesh = plsc.VectorSubcoreMesh(core_axis_name="c", subcore_axis_name="s")

    @functools.partial(
        pl.kernel, mesh=mesh,
        out_type=jax.ShapeDtypeStruct((s, d), jnp.float32),
        scratch_types=[
            pltpu.VMEM((rpw,), jnp.int32),
            pltpu.VMEM((rpw, d), jnp.float32),
            pltpu.SemaphoreType.DMA,
        ])
    def k(src_hbm, idx_hbm, out_hbm, idx_v, rows_v, sem):
        wid = lax.axis_index("s") * 2 + lax.axis_index("c")
        base = wid * rpw
        pltpu.sync_copy(idx_hbm.at[pl.ds(base, rpw)], idx_v)
        pltpu.sync_copy(src_hbm.at[pl.ds(base, rpw)], rows_v)
        pltpu.async_copy(rows_v, out_hbm.at[idx_v], sem).wait()

    return k(src, pos)


def _sc_gather_rows(src, pos):
    s, d = src.shape
    rpw = s // _NW
    mesh = plsc.VectorSubcoreMesh(core_axis_name="c", subcore_axis_name="s")

    @functools.partial(
        pl.kernel, mesh=mesh,
        out_type=jax.ShapeDtypeStruct((s, d), jnp.float32),
        scratch_types=[
            pltpu.VMEM((rpw,), jnp.int32),
            pltpu.VMEM((rpw, d), jnp.float32),
            pltpu.SemaphoreType.DMA,
        ])
    def k(src_hbm, idx_hbm, out_hbm, idx_v, rows_v, sem):
        wid = lax.axis_index("s") * 2 + lax.axis_index("c")
        base = wid * rpw
        pltpu.sync_copy(idx_hbm.at[pl.ds(base, rpw)], idx_v)
        pltpu.async_copy(src_hbm.at[idx_v], rows_v, sem).wait()
        pltpu.sync_copy(rows_v, out_hbm.at[pl.ds(base, rpw)])

    return k(src, pos)


def kernel(hidden_states, W_router, wi, wo):
    b, s, d = hidden_states.shape
    e = W_router.shape[-1]
    x = hidden_states.reshape(b * s, d)

    logits, idx2, x_scaled = _route(x, W_router)
    eidx = idx2[:, 0]

    num_tiles = (b * s) // _T
    pos, tw, ew, lo, hi, first = _dispatch_meta(eidx, e, num_tiles)

    xs = _sc_scatter_rows(x_scaled, pos)
    ys = _grouped_ffn(xs, wi, wo, (tw, ew, lo, hi, first))
    out = _sc_gather_rows(ys, pos)

    return (out.reshape(b, s, d),
            logits.reshape(b, s, e),
            eidx.reshape(b, s))
